# baseline (speedup 1.0000x reference)
import functools

import jax
import jax.numpy as jnp
from jax import lax
from jax.experimental import pallas as pl
from jax.experimental.pallas import tpu as pltpu
from jax.experimental.pallas import tpu_sc as plsc

N = 10000
E = 160000
F_IN = 256
F_HID = 512
EMB_IN = 32
F_OUT = 64
GD_STEPS = 5
LR = 0.01

NP = 10240
EP = 163840
NC, NS, L = 2, 16, 16
CHUNK = 128
CPT0 = 52
CPT1 = 28
CPTMAX = max(CPT0, CPT1)
R_PAD = NS * CPT0 + NS * CPT1 + (CPTMAX - min(CPT0, CPT1))
ROWS_PT = NP // NS

RB = 1000


def _mlp1_body(x_ref, w1, b1, w2, b2, w3, b3, o_ref):
    h = jnp.maximum(x_ref[...] @ w1[...] + b1[...], 0.0)
    h = jnp.maximum(h @ w2[...] + b2[...], 0.0)
    o_ref[...] = h @ w3[...] + b3[...]


def _mlp2_body(h_ref, e_ref, w1h, w1e, b1, w2, b2, w3, b3, o_ref):
    z = h_ref[...] @ w1h[...] + e_ref[...] @ w1e[...] + b1[...]
    z = jnp.maximum(z, 0.0)
    z = jnp.maximum(z @ w2[...] + b2[...], 0.0)
    o_ref[...] = z @ w3[...] + b3[...]


def _full(shape):
    return pl.BlockSpec(shape, lambda i: tuple(0 for _ in shape))


def _mlp1(x_feat, W1, b1, W2, b2, W3, b3):
    return pl.pallas_call(
        _mlp1_body,
        grid=(N // RB,),
        in_specs=[
            pl.BlockSpec((RB, F_IN), lambda i: (i, 0)),
            _full((F_IN, F_HID)), _full((1, F_HID)),
            _full((F_HID, F_HID)), _full((1, F_HID)),
            _full((F_HID, F_HID)), _full((1, F_HID)),
        ],
        out_specs=pl.BlockSpec((RB, F_HID), lambda i: (i, 0)),
        out_shape=jax.ShapeDtypeStruct((N, F_HID), jnp.float32),
    )(x_feat, W1, b1.reshape(1, -1), W2, b2.reshape(1, -1), W3, b3.reshape(1, -1))


def _mlp2(h, emb, W1, b1, W2, b2, W3, b3):
    W1h, W1e = W1[:F_HID], W1[F_HID:]
    return pl.pallas_call(
        _mlp2_body,
        grid=(N // RB,),
        in_specs=[
            pl.BlockSpec((RB, F_HID), lambda i: (i, 0)),
            pl.BlockSpec((RB, EMB_IN), lambda i: (i, 0)),
            _full((F_HID, F_HID)), _full((EMB_IN, F_HID)), _full((1, F_HID)),
            _full((F_HID, F_HID)), _full((1, F_HID)),
            _full((F_HID, F_OUT)), _full((1, F_OUT)),
        ],
        out_specs=pl.BlockSpec((RB, F_OUT), lambda i: (i, 0)),
        out_shape=jax.ShapeDtypeStruct((N, F_OUT), jnp.float32),
    )(h, emb, W1h, W1e, b1.reshape(1, -1), W2, b2.reshape(1, -1), W3, b3.reshape(1, -1))


def _xq0_body(x_ref, q_ref, o_ref):
    o_ref[...] = x_ref[...] @ q_ref[...]


def _xq0(emb_pad, Q):
    return pl.pallas_call(
        _xq0_body,
        out_shape=jax.ShapeDtypeStruct((NP, EMB_IN), jnp.float32),
    )(emb_pad, Q)


def _update_body(e_ref, aa_ref, am_ref, w1_ref, w2_ref, q_ref, eo_ref, xo_ref):
    aa = aa_ref[0] + aa_ref[1]
    am = am_ref[0] + am_ref[1]
    e = e_ref[...] - LR * (am @ w1_ref[...]) + LR * (aa @ w2_ref[...])
    eo_ref[...] = e
    xo_ref[...] = e @ q_ref[...]


def _update(emb_pad, acc_a, acc_m, W1t, W2t, Q):
    return pl.pallas_call(
        _update_body,
        out_shape=(jax.ShapeDtypeStruct((NP, EMB_IN), jnp.float32),
                   jax.ShapeDtypeStruct((NP, EMB_IN), jnp.float32)),
    )(emb_pad, acc_a, acc_m, W1t, W2t, Q)


def _gd_body(emb_h, xq_h, src_h, dst_h, src2_h, dst2_h, mask_h,
             acca_h, accm_h,
             srcv, dstv, src2v, dst2v, maskv, ga, gs, gd, msg, zsrc,
             acca_s, accm_s, sga, sgs, sgd, ssa, ssm):
    cid = lax.axis_index("c")
    sid = lax.axis_index("s")
    erow = jnp.where(cid == 0, sid * CPT0, NS * CPT0 + sid * CPT1)
    npairs = jnp.where(cid == 0, CPT0 // 2, CPT1 // 2)
    nbase = sid * ROWS_PT

    zero16 = jnp.zeros((L,), jnp.float32)
    for r in range(CHUNK):
        zsrc[r, pl.ds(0, L)] = zero16
        zsrc[r, pl.ds(L, L)] = zero16
    for k in range(ROWS_PT // CHUNK):
        pltpu.sync_copy(zsrc, acca_s.at[pl.ds(nbase + k * CHUNK, CHUNK)])
        pltpu.sync_copy(zsrc, accm_s.at[pl.ds(nbase + k * CHUNK, CHUNK)])
    plsc.subcore_barrier()

    pltpu.sync_copy(src_h.at[pl.ds(erow, CPTMAX)], srcv)
    pltpu.sync_copy(dst_h.at[pl.ds(erow, CPTMAX)], dstv)
    pltpu.sync_copy(src2_h.at[pl.ds(erow, CPTMAX)], src2v)
    pltpu.sync_copy(dst2_h.at[pl.ds(erow, CPTMAX)], dst2v)
    pltpu.sync_copy(mask_h.at[pl.ds(erow * CHUNK, CPTMAX * CHUNK)], maskv)

    perms = [lax.iota(jnp.int32, L) ^ k for k in (1, 2, 4, 8)]

    def issue_gathers(c, p):
        pltpu.async_copy(xq_h.at[srcv.at[c]], ga.at[p], sga.at[p])
        pltpu.async_copy(xq_h.at[src2v.at[c]], gs.at[p], sgs.at[p])
        pltpu.async_copy(emb_h.at[dst2v.at[c]], gd.at[p], sgd.at[p])

    def wait_gathers(c, p):
        pltpu.make_async_copy(xq_h.at[srcv.at[c]], ga.at[p], sga.at[p]).wait()
        pltpu.make_async_copy(xq_h.at[src2v.at[c]], gs.at[p], sgs.at[p]).wait()
        pltpu.make_async_copy(emb_h.at[dst2v.at[c]], gd.at[p], sgd.at[p]).wait()

    def wait_scatter_a(c, p):
        pltpu.make_async_copy(ga.at[p], acca_s.at[dstv.at[c]], ssa.at[p]).wait()

    def wait_scatter_m(c, p):
        pltpu.make_async_copy(msg.at[p], accm_s.at[dst2v.at[c]], ssm.at[p]).wait()

    def compute_msg(c, p):
        gsp, gdp, msgp = gs.at[p], gd.at[p], msg.at[p]

        def grp_body(g, _):
            mk16 = maskv[pl.ds(c * CHUNK + g * L, L)]
            for i in range(L):
                row = g * L + i
                a0 = gdp[row, pl.ds(0, L)]
                a1 = gdp[row, pl.ds(L, L)]
                b0 = gsp[row, pl.ds(0, L)]
                b1 = gsp[row, pl.ds(L, L)]
                s = a0 * b0 + a1 * b1
                for pe in perms:
                    s = s + s.at[pe].get(mode="promise_in_bounds")
                mi = mk16.at[jnp.full((L,), i, jnp.int32)].get(
                    mode="promise_in_bounds")
                sm = s * mi
                msgp[row, pl.ds(0, L)] = sm * b0
                msgp[row, pl.ds(L, L)] = sm * b1
            return 0

        lax.fori_loop(0, CHUNK // L, grp_body, 0)

    def issue_gathers_sd(c, p):
        pltpu.async_copy(xq_h.at[src2v.at[c]], gs.at[p], sgs.at[p])
        pltpu.async_copy(emb_h.at[dst2v.at[c]], gd.at[p], sgd.at[p])

    def issue_gather_a(c, p):
        pltpu.async_copy(xq_h.at[srcv.at[c]], ga.at[p], sga.at[p])

    def chunk_work(c, ph, cc):
        q = 1 - ph
        if ph == 1:
            @pl.when(cc < npairs - 1)
            def _():
                issue_gathers_sd(c + 1, q)
        else:
            issue_gathers_sd(c + 1, q)
        if ph == 0:
            @pl.when(cc > 0)
            def _():
                wait_scatter_a(c, q)

            @pl.when(cc < npairs)
            def _():
                issue_gather_a(c + 1, q)
        else:
            wait_scatter_a(c, q)

            @pl.when(cc < npairs - 1)
            def _():
                issue_gather_a(c + 1, q)
        wait_gathers(c, ph)
        pltpu.async_copy(ga.at[ph], acca_s.at[dstv.at[c]], ssa.at[ph], add=True)
        @pl.when(cc > 0)
        def _():
            wait_scatter_m(c, ph)
        compute_msg(c, ph)
        pltpu.async_copy(msg.at[ph], accm_s.at[dst2v.at[c]], ssm.at[ph], add=True)

    issue_gathers(0, 0)

    def pair_body(cc, _):
        chunk_work(2 * cc, 0, cc)
        chunk_work(2 * cc + 1, 1, cc)
        return 0

    lax.fori_loop(0, npairs, pair_body, 0)
    wait_scatter_a(0, 1)
    wait_scatter_m(0, 0)
    wait_scatter_m(0, 1)

    plsc.subcore_barrier()
    pltpu.sync_copy(acca_s.at[pl.ds(nbase, ROWS_PT)],
                    acca_h.at[cid, pl.ds(nbase, ROWS_PT)])
    pltpu.sync_copy(accm_s.at[pl.ds(nbase, ROWS_PT)],
                    accm_h.at[cid, pl.ds(nbase, ROWS_PT)])


_gd_step = functools.partial(
    pl.kernel,
    _gd_body,
    out_type=(jax.ShapeDtypeStruct((NC, NP, EMB_IN), jnp.float32),
              jax.ShapeDtypeStruct((NC, NP, EMB_IN), jnp.float32)),
    mesh=plsc.VectorSubcoreMesh(core_axis_name="c", subcore_axis_name="s",
                                num_cores=NC, num_subcores=NS),
    compiler_params=pltpu.CompilerParams(needs_layout_passes=False,
                                         use_tc_tiling_on_sc=False),
    scratch_types=[
        pltpu.VMEM((CPTMAX, CHUNK), jnp.int32),
        pltpu.VMEM((CPTMAX, CHUNK), jnp.int32),
        pltpu.VMEM((CPTMAX, CHUNK), jnp.int32),
        pltpu.VMEM((CPTMAX, CHUNK), jnp.int32),
        pltpu.VMEM((CPTMAX * CHUNK,), jnp.float32),
        pltpu.VMEM((2, CHUNK, EMB_IN), jnp.float32),
        pltpu.VMEM((2, CHUNK, EMB_IN), jnp.float32),
        pltpu.VMEM((2, CHUNK, EMB_IN), jnp.float32),
        pltpu.VMEM((2, CHUNK, EMB_IN), jnp.float32),
        pltpu.VMEM((CHUNK, EMB_IN), jnp.float32),
        pltpu.VMEM_SHARED((NP, EMB_IN), jnp.float32),
        pltpu.VMEM_SHARED((NP, EMB_IN), jnp.float32),
        pltpu.SemaphoreType.DMA((2,)),
        pltpu.SemaphoreType.DMA((2,)),
        pltpu.SemaphoreType.DMA((2,)),
        pltpu.SemaphoreType.DMA((2,)),
        pltpu.SemaphoreType.DMA((2,)),
    ],
)()


def kernel(x_feat, x, edge_index, edge_index_2, Q, mask, W_f1, b_f1, W_f2, b_f2, W_f3, b_f3, W_c1, b_c1, W_c2, b_c2, W_c3, b_c3, gd_W1, gd_W2):
    h = _mlp1(x_feat, W_f1, b_f1, W_f2, b_f2, W_f3, b_f3)

    pad_e = jnp.full((R_PAD * CHUNK - E,), N, jnp.int32)
    src = jnp.concatenate([edge_index[0], pad_e]).reshape(R_PAD, CHUNK)
    dst = jnp.concatenate([edge_index[1], pad_e]).reshape(R_PAD, CHUNK)
    src2 = jnp.concatenate([edge_index_2[0], pad_e]).reshape(R_PAD, CHUNK)
    dst2 = jnp.concatenate([edge_index_2[1], pad_e]).reshape(R_PAD, CHUNK)
    mask_p = jnp.concatenate(
        [mask, jnp.zeros((R_PAD * CHUNK - E,), jnp.float32)])

    emb = jnp.concatenate([x, jnp.zeros((NP - N, EMB_IN), jnp.float32)])
    xq = _xq0(emb, Q)
    for t in range(GD_STEPS):
        acc_a, acc_m = _gd_step(emb, xq, src, dst, src2, dst2, mask_p)
        emb, xq = _update(emb, acc_a, acc_m, gd_W1[t], gd_W2[t], Q)

    emb_out = emb[:N]
    out = _mlp2(h, emb_out, W_c1, b_c1, W_c2, b_c2, W_c3, b_c3)
    return out, emb_out


# speedup vs baseline: 1.0737x; 1.0737x over previous
import functools

import jax
import jax.numpy as jnp
from jax import lax
from jax.experimental import pallas as pl
from jax.experimental.pallas import tpu as pltpu
from jax.experimental.pallas import tpu_sc as plsc

N = 10000
E = 160000
F_IN = 256
F_HID = 512
EMB_IN = 32
F_OUT = 64
GD_STEPS = 5
LR = 0.01

NP = 10240
EP = 163840
NC, NS, L = 2, 16, 16
CHUNK = 128
CPT0 = 58
CPT1 = 22
CPTMAX = max(CPT0, CPT1)
R_PAD = NS * CPT0 + NS * CPT1 + (CPTMAX - min(CPT0, CPT1))
ROWS_PT = NP // NS

RB = 1000


def _mlp1_body(x_ref, w1, b1, w2, b2, w3, b3, o_ref):
    h = jnp.maximum(x_ref[...] @ w1[...] + b1[...], 0.0)
    h = jnp.maximum(h @ w2[...] + b2[...], 0.0)
    o_ref[...] = h @ w3[...] + b3[...]


def _mlp2_body(h_ref, e_ref, w1h, w1e, b1, w2, b2, w3, b3, o_ref):
    z = h_ref[...] @ w1h[...] + e_ref[...] @ w1e[...] + b1[...]
    z = jnp.maximum(z, 0.0)
    z = jnp.maximum(z @ w2[...] + b2[...], 0.0)
    o_ref[...] = z @ w3[...] + b3[...]


def _full(shape):
    return pl.BlockSpec(shape, lambda i: tuple(0 for _ in shape))


def _mlp1(x_feat, W1, b1, W2, b2, W3, b3):
    return pl.pallas_call(
        _mlp1_body,
        grid=(N // RB,),
        in_specs=[
            pl.BlockSpec((RB, F_IN), lambda i: (i, 0)),
            _full((F_IN, F_HID)), _full((1, F_HID)),
            _full((F_HID, F_HID)), _full((1, F_HID)),
            _full((F_HID, F_HID)), _full((1, F_HID)),
        ],
        out_specs=pl.BlockSpec((RB, F_HID), lambda i: (i, 0)),
        out_shape=jax.ShapeDtypeStruct((N, F_HID), jnp.float32),
    )(x_feat, W1, b1.reshape(1, -1), W2, b2.reshape(1, -1), W3, b3.reshape(1, -1))


def _mlp2(h, emb, W1, b1, W2, b2, W3, b3):
    W1h, W1e = W1[:F_HID], W1[F_HID:]
    return pl.pallas_call(
        _mlp2_body,
        grid=(N // RB,),
        in_specs=[
            pl.BlockSpec((RB, F_HID), lambda i: (i, 0)),
            pl.BlockSpec((RB, EMB_IN), lambda i: (i, 0)),
            _full((F_HID, F_HID)), _full((EMB_IN, F_HID)), _full((1, F_HID)),
            _full((F_HID, F_HID)), _full((1, F_HID)),
            _full((F_HID, F_OUT)), _full((1, F_OUT)),
        ],
        out_specs=pl.BlockSpec((RB, F_OUT), lambda i: (i, 0)),
        out_shape=jax.ShapeDtypeStruct((N, F_OUT), jnp.float32),
    )(h, emb, W1h, W1e, b1.reshape(1, -1), W2, b2.reshape(1, -1), W3, b3.reshape(1, -1))


def _xq0_body(x_ref, q_ref, o_ref):
    o_ref[...] = x_ref[...] @ q_ref[...]


def _xq0(emb_pad, Q):
    return pl.pallas_call(
        _xq0_body,
        out_shape=jax.ShapeDtypeStruct((NP, EMB_IN), jnp.float32),
    )(emb_pad, Q)


def _update_body(e_ref, aa_ref, am_ref, w1_ref, w2_ref, q_ref, eo_ref, xo_ref):
    aa = aa_ref[0] + aa_ref[1]
    am = am_ref[0] + am_ref[1]
    e = e_ref[...] - LR * (am @ w1_ref[...]) + LR * (aa @ w2_ref[...])
    eo_ref[...] = e
    xo_ref[...] = e @ q_ref[...]


def _update(emb_pad, acc_a, acc_m, W1t, W2t, Q):
    return pl.pallas_call(
        _update_body,
        out_shape=(jax.ShapeDtypeStruct((NP, EMB_IN), jnp.float32),
                   jax.ShapeDtypeStruct((NP, EMB_IN), jnp.float32)),
    )(emb_pad, acc_a, acc_m, W1t, W2t, Q)


def _gd_body(emb_h, xq_h, src_h, dst_h, src2_h, dst2_h, mask_h,
             acca_h, accm_h,
             srcv, dstv, src2v, dst2v, maskv, ga, gs, gd, msg, zsrc,
             acca_s, accm_s, sga, sgs, sgd, ssa, ssm):
    cid = lax.axis_index("c")
    sid = lax.axis_index("s")
    erow = jnp.where(cid == 0, sid * CPT0, NS * CPT0 + sid * CPT1)
    npairs = jnp.where(cid == 0, CPT0 // 2, CPT1 // 2)
    nbase = sid * ROWS_PT

    zero16 = jnp.zeros((L,), jnp.float32)
    for r in range(CHUNK):
        zsrc[r, pl.ds(0, L)] = zero16
        zsrc[r, pl.ds(L, L)] = zero16
    for k in range(ROWS_PT // CHUNK):
        pltpu.sync_copy(zsrc, acca_s.at[pl.ds(nbase + k * CHUNK, CHUNK)])
        pltpu.sync_copy(zsrc, accm_s.at[pl.ds(nbase + k * CHUNK, CHUNK)])
    plsc.subcore_barrier()

    pltpu.sync_copy(src_h.at[pl.ds(erow, CPTMAX)], srcv)
    pltpu.sync_copy(dst_h.at[pl.ds(erow, CPTMAX)], dstv)
    pltpu.sync_copy(src2_h.at[pl.ds(erow, CPTMAX)], src2v)
    pltpu.sync_copy(dst2_h.at[pl.ds(erow, CPTMAX)], dst2v)
    pltpu.sync_copy(mask_h.at[pl.ds(erow * CHUNK, CPTMAX * CHUNK)], maskv)

    perms = [lax.iota(jnp.int32, L) ^ k for k in (1, 2, 4, 8)]

    def issue_gathers(c, p):
        pltpu.async_copy(xq_h.at[srcv.at[c]], ga.at[p], sga.at[p])
        pltpu.async_copy(xq_h.at[src2v.at[c]], gs.at[p], sgs.at[p])
        pltpu.async_copy(emb_h.at[dst2v.at[c]], gd.at[p], sgd.at[p])

    def wait_gathers(c, p):
        pltpu.make_async_copy(xq_h.at[srcv.at[c]], ga.at[p], sga.at[p]).wait()
        pltpu.make_async_copy(xq_h.at[src2v.at[c]], gs.at[p], sgs.at[p]).wait()
        pltpu.make_async_copy(emb_h.at[dst2v.at[c]], gd.at[p], sgd.at[p]).wait()

    def wait_scatter_a(c, p):
        pltpu.make_async_copy(ga.at[p], acca_s.at[dstv.at[c]], ssa.at[p]).wait()

    def wait_scatter_m(c, p):
        pltpu.make_async_copy(msg.at[p], accm_s.at[dst2v.at[c]], ssm.at[p]).wait()

    def compute_msg(c, p):
        gsp, gdp, msgp = gs.at[p], gd.at[p], msg.at[p]

        def grp_body(g, _):
            mk16 = maskv[pl.ds(c * CHUNK + g * L, L)]
            for i in range(L):
                row = g * L + i
                a0 = gdp[row, pl.ds(0, L)]
                a1 = gdp[row, pl.ds(L, L)]
                b0 = gsp[row, pl.ds(0, L)]
                b1 = gsp[row, pl.ds(L, L)]
                s = a0 * b0 + a1 * b1
                for pe in perms:
                    s = s + s.at[pe].get(mode="promise_in_bounds")
                mi = mk16.at[jnp.full((L,), i, jnp.int32)].get(
                    mode="promise_in_bounds")
                sm = s * mi
                msgp[row, pl.ds(0, L)] = sm * b0
                msgp[row, pl.ds(L, L)] = sm * b1
            return 0

        lax.fori_loop(0, CHUNK // L, grp_body, 0)

    def issue_gathers_sd(c, p):
        pltpu.async_copy(xq_h.at[src2v.at[c]], gs.at[p], sgs.at[p])
        pltpu.async_copy(emb_h.at[dst2v.at[c]], gd.at[p], sgd.at[p])

    def issue_gather_a(c, p):
        pltpu.async_copy(xq_h.at[srcv.at[c]], ga.at[p], sga.at[p])

    def chunk_work(c, ph, cc):
        q = 1 - ph
        if ph == 1:
            @pl.when(cc < npairs - 1)
            def _():
                issue_gathers_sd(c + 1, q)
        else:
            issue_gathers_sd(c + 1, q)
        if ph == 0:
            @pl.when(cc > 0)
            def _():
                wait_scatter_a(c, q)

            @pl.when(cc < npairs)
            def _():
                issue_gather_a(c + 1, q)
        else:
            wait_scatter_a(c, q)

            @pl.when(cc < npairs - 1)
            def _():
                issue_gather_a(c + 1, q)
        wait_gathers(c, ph)
        pltpu.async_copy(ga.at[ph], acca_s.at[dstv.at[c]], ssa.at[ph], add=True)
        @pl.when(cc > 0)
        def _():
            wait_scatter_m(c, ph)
        compute_msg(c, ph)
        pltpu.async_copy(msg.at[ph], accm_s.at[dst2v.at[c]], ssm.at[ph], add=True)

    issue_gathers(0, 0)

    def pair_body(cc, _):
        chunk_work(2 * cc, 0, cc)
        chunk_work(2 * cc + 1, 1, cc)
        return 0

    lax.fori_loop(0, npairs, pair_body, 0)
    wait_scatter_a(0, 1)
    wait_scatter_m(0, 0)
    wait_scatter_m(0, 1)

    plsc.subcore_barrier()
    pltpu.sync_copy(acca_s.at[pl.ds(nbase, ROWS_PT)],
                    acca_h.at[cid, pl.ds(nbase, ROWS_PT)])
    pltpu.sync_copy(accm_s.at[pl.ds(nbase, ROWS_PT)],
                    accm_h.at[cid, pl.ds(nbase, ROWS_PT)])


_gd_step = functools.partial(
    pl.kernel,
    _gd_body,
    out_type=(jax.ShapeDtypeStruct((NC, NP, EMB_IN), jnp.float32),
              jax.ShapeDtypeStruct((NC, NP, EMB_IN), jnp.float32)),
    mesh=plsc.VectorSubcoreMesh(core_axis_name="c", subcore_axis_name="s",
                                num_cores=NC, num_subcores=NS),
    compiler_params=pltpu.CompilerParams(needs_layout_passes=False,
                                         use_tc_tiling_on_sc=False),
    scratch_types=[
        pltpu.VMEM((CPTMAX, CHUNK), jnp.int32),
        pltpu.VMEM((CPTMAX, CHUNK), jnp.int32),
        pltpu.VMEM((CPTMAX, CHUNK), jnp.int32),
        pltpu.VMEM((CPTMAX, CHUNK), jnp.int32),
        pltpu.VMEM((CPTMAX * CHUNK,), jnp.float32),
        pltpu.VMEM((2, CHUNK, EMB_IN), jnp.float32),
        pltpu.VMEM((2, CHUNK, EMB_IN), jnp.float32),
        pltpu.VMEM((2, CHUNK, EMB_IN), jnp.float32),
        pltpu.VMEM((2, CHUNK, EMB_IN), jnp.float32),
        pltpu.VMEM((CHUNK, EMB_IN), jnp.float32),
        pltpu.VMEM_SHARED((NP, EMB_IN), jnp.float32),
        pltpu.VMEM_SHARED((NP, EMB_IN), jnp.float32),
        pltpu.SemaphoreType.DMA((2,)),
        pltpu.SemaphoreType.DMA((2,)),
        pltpu.SemaphoreType.DMA((2,)),
        pltpu.SemaphoreType.DMA((2,)),
        pltpu.SemaphoreType.DMA((2,)),
    ],
)()


def kernel(x_feat, x, edge_index, edge_index_2, Q, mask, W_f1, b_f1, W_f2, b_f2, W_f3, b_f3, W_c1, b_c1, W_c2, b_c2, W_c3, b_c3, gd_W1, gd_W2):
    h = _mlp1(x_feat, W_f1, b_f1, W_f2, b_f2, W_f3, b_f3)

    pad_e = jnp.full((R_PAD * CHUNK - E,), N, jnp.int32)
    src = jnp.concatenate([edge_index[0], pad_e]).reshape(R_PAD, CHUNK)
    dst = jnp.concatenate([edge_index[1], pad_e]).reshape(R_PAD, CHUNK)
    src2 = jnp.concatenate([edge_index_2[0], pad_e]).reshape(R_PAD, CHUNK)
    dst2 = jnp.concatenate([edge_index_2[1], pad_e]).reshape(R_PAD, CHUNK)
    mask_p = jnp.concatenate(
        [mask, jnp.zeros((R_PAD * CHUNK - E,), jnp.float32)])

    emb = jnp.concatenate([x, jnp.zeros((NP - N, EMB_IN), jnp.float32)])
    xq = _xq0(emb, Q)
    for t in range(GD_STEPS):
        acc_a, acc_m = _gd_step(emb, xq, src, dst, src2, dst2, mask_p)
        emb, xq = _update(emb, acc_a, acc_m, gd_W1[t], gd_W2[t], Q)

    emb_out = emb[:N]
    out = _mlp2(h, emb_out, W_c1, b_c1, W_c2, b_c2, W_c3, b_c3)
    return out, emb_out
